# Initial kernel scaffold; baseline (speedup 1.0000x reference)
import jax
import jax.numpy as jnp
from jax.experimental import pallas as pl


def kernel(input_ids, word_embedding_table):
    raise NotImplementedError("write your pallas kernel here")


# speedup vs baseline: 1.6453x; 1.6453x over previous
import functools

import jax
import jax.numpy as jnp
from jax import lax
from jax.experimental import pallas as pl
from jax.experimental.pallas import tpu as pltpu
from jax.experimental.pallas import tpu_sc as plsc

D_MODEL = 768
CHUNK = 64
NBUF = 2
NC, NS = 2, 16
NW = NC * NS


@functools.lru_cache(maxsize=None)
def _make_gather(total: int):
    rows_per_w = total // NW
    chunks_per_w = rows_per_w // CHUNK
    mesh = plsc.VectorSubcoreMesh(core_axis_name="c", subcore_axis_name="s")

    @functools.partial(
        pl.kernel,
        out_type=jax.ShapeDtypeStruct((total, D_MODEL), jnp.float32),
        mesh=mesh,
        scratch_types=[
            pltpu.VMEM((chunks_per_w, CHUNK), jnp.int32),
            [pltpu.VMEM((CHUNK, D_MODEL), jnp.float32) for _ in range(NBUF)],
            [pltpu.SemaphoreType.DMA for _ in range(NBUF)],
            [pltpu.SemaphoreType.DMA for _ in range(NBUF)],
        ],
    )
    def k(ids_hbm, table_hbm, out_hbm, idx_v, bufs, gsems, wsems):
        wid = lax.axis_index("s") * NC + lax.axis_index("c")
        pltpu.sync_copy(ids_hbm.at[pl.ds(wid * chunks_per_w, chunks_per_w)], idx_v)
        row_base = wid * rows_per_w

        def gather(c, b):
            return pltpu.async_copy(table_hbm.at[idx_v.at[c]], bufs[b], gsems[b])

        def write(c, b):
            dst = out_hbm.at[pl.ds(row_base + c * CHUNK, CHUNK)]
            return pltpu.async_copy(bufs[b], dst, wsems[b])

        g = [None] * NBUF
        w = [None] * NBUF
        g[0] = gather(0, 0)
        for c in range(chunks_per_w):
            b = c % NBUF
            g[b].wait()
            nb = (c + 1) % NBUF
            if c + 1 < chunks_per_w:
                if w[nb] is not None:
                    w[nb].wait()
                g[nb] = gather(c + 1, nb)
            w[b] = write(c, b)
        for h in w:
            if h is not None:
                h.wait()

    return k


def kernel(input_ids, word_embedding_table):
    bsz, seq = input_ids.shape
    total = bsz * seq
    ids = input_ids.reshape(total // CHUNK, CHUNK).astype(jnp.int32)
    out = _make_gather(total)(ids, word_embedding_table)
    return out.reshape(bsz, seq, D_MODEL)


# speedup vs baseline: 1.6647x; 1.0118x over previous
import functools

import jax
import jax.numpy as jnp
from jax import lax
from jax.experimental import pallas as pl
from jax.experimental.pallas import tpu as pltpu
from jax.experimental.pallas import tpu_sc as plsc

D_MODEL = 768
CHUNK = 32
NBUF = 4
NC, NS = 2, 16
NW = NC * NS


@functools.lru_cache(maxsize=None)
def _make_gather(total: int):
    rows_per_w = total // NW
    chunks_per_w = rows_per_w // CHUNK
    mesh = plsc.VectorSubcoreMesh(core_axis_name="c", subcore_axis_name="s")

    @functools.partial(
        pl.kernel,
        out_type=jax.ShapeDtypeStruct((total, D_MODEL), jnp.float32),
        mesh=mesh,
        scratch_types=[
            pltpu.VMEM((chunks_per_w, CHUNK), jnp.int32),
            [pltpu.VMEM((CHUNK, D_MODEL), jnp.float32) for _ in range(NBUF)],
            [pltpu.SemaphoreType.DMA for _ in range(NBUF)],
            [pltpu.SemaphoreType.DMA for _ in range(NBUF)],
        ],
    )
    def k(ids_hbm, table_hbm, out_hbm, idx_v, bufs, gsems, wsems):
        wid = lax.axis_index("s") * NC + lax.axis_index("c")
        pltpu.sync_copy(ids_hbm.at[pl.ds(wid * chunks_per_w, chunks_per_w)], idx_v)
        row_base = wid * rows_per_w

        def gather(c, b):
            return pltpu.async_copy(table_hbm.at[idx_v.at[c]], bufs[b], gsems[b])

        def write(c, b):
            dst = out_hbm.at[pl.ds(row_base + c * CHUNK, CHUNK)]
            return pltpu.async_copy(bufs[b], dst, wsems[b])

        prime = max(1, NBUF // 2)
        g = [None] * NBUF
        w = [None] * NBUF
        for c in range(prime):
            g[c] = gather(c, c)
        for c in range(chunks_per_w):
            b = c % NBUF
            g[b].wait()
            nxt = c + prime
            if nxt < chunks_per_w:
                nb = nxt % NBUF
                if w[nb] is not None:
                    w[nb].wait()
                    w[nb] = None
                g[nb] = gather(nxt, nb)
            w[b] = write(c, b)
        for h in w:
            if h is not None:
                h.wait()

    return k


def kernel(input_ids, word_embedding_table):
    bsz, seq = input_ids.shape
    total = bsz * seq
    ids = input_ids.reshape(total // CHUNK, CHUNK).astype(jnp.int32)
    out = _make_gather(total)(ids, word_embedding_table)
    return out.reshape(bsz, seq, D_MODEL)


# speedup vs baseline: 1.6795x; 1.0089x over previous
import functools

import jax
import jax.numpy as jnp
from jax import lax
from jax.experimental import pallas as pl
from jax.experimental.pallas import tpu as pltpu
from jax.experimental.pallas import tpu_sc as plsc

D_MODEL = 768
CHUNK = 32
NBUF = 4
NC, NS = 2, 16
NW = NC * NS


@functools.lru_cache(maxsize=None)
def _make_gather(bsz: int, seq: int):
    total = bsz * seq
    rows_per_w = total // NW
    chunks_per_w = rows_per_w // CHUNK
    w_per_row = seq // rows_per_w
    mesh = plsc.VectorSubcoreMesh(core_axis_name="c", subcore_axis_name="s")

    @functools.partial(
        pl.kernel,
        out_type=jax.ShapeDtypeStruct((total, D_MODEL), jnp.float32),
        mesh=mesh,
        scratch_types=[
            pltpu.VMEM((rows_per_w,), jnp.int32),
            [pltpu.VMEM((CHUNK, D_MODEL), jnp.float32) for _ in range(NBUF)],
            [pltpu.SemaphoreType.DMA for _ in range(NBUF)],
            [pltpu.SemaphoreType.DMA for _ in range(NBUF)],
        ],
    )
    def k(ids_hbm, table_hbm, out_hbm, idx_v, bufs, gsems, wsems):
        wid = lax.axis_index("s") * NC + lax.axis_index("c")
        src = ids_hbm.at[wid // w_per_row, pl.ds((wid % w_per_row) * rows_per_w, rows_per_w)]
        pltpu.sync_copy(src, idx_v)
        row_base = wid * rows_per_w

        def gather(c, b):
            idx = idx_v.at[pl.ds(c * CHUNK, CHUNK)]
            return pltpu.async_copy(table_hbm.at[idx], bufs[b], gsems[b])

        def write(c, b):
            dst = out_hbm.at[pl.ds(row_base + c * CHUNK, CHUNK)]
            return pltpu.async_copy(bufs[b], dst, wsems[b])

        prime = max(1, NBUF // 2)
        g = [None] * NBUF
        w = [None] * NBUF
        for c in range(prime):
            g[c] = gather(c, c)
        for c in range(chunks_per_w):
            b = c % NBUF
            g[b].wait()
            nxt = c + prime
            if nxt < chunks_per_w:
                nb = nxt % NBUF
                if w[nb] is not None:
                    w[nb].wait()
                    w[nb] = None
                g[nb] = gather(nxt, nb)
            w[b] = write(c, b)
        for h in w:
            if h is not None:
                h.wait()

    return k


def kernel(input_ids, word_embedding_table):
    bsz, seq = input_ids.shape
    ids = input_ids.astype(jnp.int32)
    out = _make_gather(bsz, seq)(ids, word_embedding_table)
    return out.reshape(bsz, seq, D_MODEL)


# speedup vs baseline: 1.6812x; 1.0010x over previous
import functools

import jax
import jax.numpy as jnp
from jax import lax
from jax.experimental import pallas as pl
from jax.experimental.pallas import tpu as pltpu
from jax.experimental.pallas import tpu_sc as plsc

D_MODEL = 768
CHUNK = 32
NBUF = 4
NC, NS = 2, 16
NW = NC * NS


@functools.lru_cache(maxsize=None)
def _make_gather(bsz: int, seq: int):
    total = bsz * seq
    rows_per_w = total // NW
    chunks_per_w = rows_per_w // CHUNK
    w_per_row = seq // rows_per_w
    mesh = plsc.VectorSubcoreMesh(core_axis_name="c", subcore_axis_name="s")

    @functools.partial(
        pl.kernel,
        out_type=jax.ShapeDtypeStruct((total, D_MODEL), jnp.float32),
        mesh=mesh,
        scratch_types=[
            pltpu.VMEM((rows_per_w,), jnp.int32),
            [pltpu.VMEM((CHUNK, D_MODEL), jnp.float32) for _ in range(NBUF)],
            [pltpu.SemaphoreType.DMA for _ in range(NBUF)],
            [pltpu.SemaphoreType.DMA for _ in range(NBUF)],
        ],
    )
    def k(ids_hbm, table_hbm, out_hbm, idx_v, bufs, gsems, wsems):
        wid = lax.axis_index("s") * NC + lax.axis_index("c")
        src = ids_hbm.at[wid // w_per_row, pl.ds((wid % w_per_row) * rows_per_w, rows_per_w)]
        pltpu.sync_copy(src, idx_v)
        row_base = wid * rows_per_w

        def gather(c, b):
            idx = idx_v.at[pl.ds(c * CHUNK, CHUNK)]
            return pltpu.async_copy(table_hbm.at[idx], bufs[b], gsems[b])

        def write(c, b):
            dst = out_hbm.at[pl.ds(row_base + c * CHUNK, CHUNK)]
            return pltpu.async_copy(bufs[b], dst, wsems[b])

        prime = 3
        g = [None] * NBUF
        w = [None] * NBUF
        for c in range(prime):
            g[c] = gather(c, c)
        for c in range(chunks_per_w):
            b = c % NBUF
            g[b].wait()
            nxt = c + prime
            if nxt < chunks_per_w:
                nb = nxt % NBUF
                if w[nb] is not None:
                    w[nb].wait()
                    w[nb] = None
                g[nb] = gather(nxt, nb)
            w[b] = write(c, b)
        for h in w:
            if h is not None:
                h.wait()

    return k


def kernel(input_ids, word_embedding_table):
    bsz, seq = input_ids.shape
    ids = input_ids.astype(jnp.int32)
    out = _make_gather(bsz, seq)(ids, word_embedding_table)
    return out.reshape(bsz, seq, D_MODEL)
